# baseline (speedup 1.0000x reference)
import jax
import jax.numpy as jnp
from jax.experimental import pallas as pl
from jax.experimental.pallas import tpu as pltpu

B = 16
H = W = 224
SIZE = 223
K = SIZE * SIZE
NPER = 256
NR = 32
NT = NPER // NR
GRID = 3 * NT


def _conv_kernel(cw_ref, x_ref, out_ref):
    c = pl.program_id(0)
    w00 = cw_ref[c, 0]
    w01 = cw_ref[c, 1]
    w10 = cw_ref[c, 2]
    w11 = cw_ref[c, 3]
    xs = x_ref[0]
    y = jnp.tanh(
        w00 * xs[:, :SIZE, :SIZE]
        + w01 * xs[:, :SIZE, 1:]
        + w10 * xs[:, 1:, :SIZE]
        + w11 * xs[:, 1:, 1:]
    )
    for r in range(SIZE):
        out_ref[0, :, r * SIZE:(r + 1) * SIZE] = y[:, r, :]


def _mm_kernel(flats_ref, wr_ref, wg_ref, wb_ref, bias_ref, out_ref):
    i = pl.program_id(0)
    for c, wref in enumerate((wr_ref, wg_ref, wb_ref)):
        @pl.when((i >= c * NT) & (i < (c + 1) * NT))
        def _(c=c, wref=wref):
            f = flats_ref[c]
            w = wref[...]
            y = jax.lax.dot_general(
                f, w, (((1,), (1,)), ((), ())),
                preferred_element_type=jnp.float32)
            out_ref[0] = jnp.tanh(y + bias_ref[0])


def _conv_flats(x, cw, interpret=False):
    return pl.pallas_call(
        _conv_kernel,
        grid=(3,),
        in_specs=[
            pl.BlockSpec(memory_space=pltpu.SMEM),
            pl.BlockSpec((1, B, H, W), lambda c: (c, 0, 0, 0)),
        ],
        out_specs=pl.BlockSpec((1, B, K), lambda c: (c, 0, 0)),
        out_shape=jax.ShapeDtypeStruct((3, B, K), jnp.float32),
        interpret=interpret,
    )(cw, x)


def _matmul(flats, fw_r, fw_g, fw_b, bias, interpret=False):
    return pl.pallas_call(
        _mm_kernel,
        grid=(GRID,),
        in_specs=[
            pl.BlockSpec((3, B, K), lambda i: (0, 0, 0)),
            pl.BlockSpec((NR, K), lambda i: (jnp.minimum(i, NT - 1), 0)),
            pl.BlockSpec((NR, K), lambda i: (jnp.clip(i - NT, 0, NT - 1), 0)),
            pl.BlockSpec((NR, K), lambda i: (jnp.clip(i - 2 * NT, 0, NT - 1), 0)),
            pl.BlockSpec((1, 1, NR), lambda i: (i, 0, 0)),
        ],
        out_specs=pl.BlockSpec((1, B, NR), lambda i: (i, 0, 0)),
        out_shape=jax.ShapeDtypeStruct((GRID, B, NR), jnp.float32),
        compiler_params=pltpu.CompilerParams(
            dimension_semantics=("arbitrary",)),
        interpret=interpret,
    )(flats, fw_r, fw_g, fw_b, bias)


def kernel(x, w_red, w_green, w_blue, fc_red_w, fc_red_b,
           fc_green_w, fc_green_b, fc_blue_w, fc_blue_b,
           interpret=False):
    cw = jnp.stack([w_red.reshape(4), w_green.reshape(4), w_blue.reshape(4)])
    flats = _conv_flats(x, cw, interpret=interpret)
    bias = jnp.concatenate([fc_red_b, fc_green_b, fc_blue_b]).reshape(GRID, 1, NR)
    tiles = _matmul(flats, fc_red_w, fc_green_w, fc_blue_w, bias,
                    interpret=interpret)
    return tiles.transpose(1, 0, 2).reshape(B, 3 * NPER)
